# baseline (speedup 1.0000x reference)
import functools
import math

import jax
import jax.numpy as jnp
from jax import lax
from jax.experimental import pallas as pl
from jax.experimental.pallas import tpu as pltpu
from jax.experimental.pallas import tpu_sc as plsc

_EPS = 0.001
_L_LO = math.log(_EPS) - math.log1p(-_EPS)
_L_HI = -_L_LO

_IN = 512
_HID = 512
_OUT = 128
_NS = 16


def _gln_body(nboth_h, side_h, offs_h, cons_h, w0_h, w1_h, out_h,
              side_v, nboth_v, offs_v, cons_v, w0s_v, w1s_v, tmp4_v,
              part_v, out_v, shr_h0, shr_h1, sh_v, sem_in, sem_w0, sem_w1):
    cid = lax.axis_index("c")
    sid = lax.axis_index("s")

    @pl.when(cid == 0)
    def _core0():
        t = sid
        b = t >> 2
        r = t & 3

        c1 = pltpu.async_copy(side_h, side_v, sem_in)
        c2 = pltpu.async_copy(nboth_h, nboth_v, sem_in)
        c3 = pltpu.async_copy(offs_h, offs_v, sem_in)
        c4 = pltpu.async_copy(cons_h, cons_v, sem_in)
        c1.wait(); c2.wait(); c3.wait(); c4.wait()

        def gbody(k, acc):
            base = k * 16
            sv = side_v[pl.ds(base, 16)]
            for j in range(16):
                acc = acc + sv[j] * nboth_v[base + j, :]
            return acc
        acc = lax.fori_loop(0, _IN // 16, gbody, jnp.zeros((16,), jnp.float32))
        bits = acc > offs_v[...]
        lane = lax.iota(jnp.int32, 16)
        pw = jnp.int32(1) << (7 - (lane & 7))
        ivp = jnp.where(bits, pw, jnp.int32(0))
        g0 = ivp[0]
        g1 = ivp[8]
        for j in range(1, 8):
            g0 = g0 + ivp[j]
            g1 = g1 + ivp[8 + j]

        row0 = pl.multiple_of(r * 128, 128)
        col0 = pl.multiple_of(b * 128, 128)
        w1r0 = pl.multiple_of(t * 32, 32)
        cw0 = pltpu.async_copy(
            w0_h.at[g0, pl.ds(row0, 136), pl.ds(col0, 128)], w0s_v, sem_w0)
        cw1 = pltpu.async_copy(
            w1_h.at[g1, pl.ds(w1r0, 40), :], w1s_v, sem_w1)

        cv = cons_v[...]
        lb0 = cv[0]
        lb1 = cv[1]

        cw0.wait()
        sb0 = jnp.where(r == 0, lb0, 0.0).astype(jnp.float32)
        p = [sb0 * w0s_v[0, pl.ds(16 * o, 16)] for o in range(8)]

        def mbody(k, carry):
            sv = side_v[pl.ds(r * 128 + k * 16, 16)]
            sv = jnp.minimum(jnp.maximum(sv, _EPS), 1.0 - _EPS)
            out = list(carry)
            for j in range(16):
                s = sv[j]
                l = k * 16 + j + 1
                for o in range(8):
                    out[o] = out[o] + s * w0s_v[l, pl.ds(16 * o, 16)]
            return tuple(out)
        p = lax.fori_loop(0, 8, mbody, tuple(p))

        for o in range(8):
            part_v[pl.ds(16 * o, 16)] = p[o]
        pltpu.sync_copy(part_v, shr_h0.at[t])
        plsc.subcore_barrier()

        pltpu.sync_copy(shr_h0.at[pl.ds(b * 4, 4), :], tmp4_v)
        c0 = r * 32
        h0a = (tmp4_v[0, pl.ds(c0, 16)] + tmp4_v[1, pl.ds(c0, 16)]
               + tmp4_v[2, pl.ds(c0, 16)] + tmp4_v[3, pl.ds(c0, 16)])
        c1 = c0 + 16
        h0b = (tmp4_v[0, pl.ds(c1, 16)] + tmp4_v[1, pl.ds(c1, 16)]
               + tmp4_v[2, pl.ds(c1, 16)] + tmp4_v[3, pl.ds(c1, 16)])
        h0a = jnp.minimum(jnp.maximum(h0a, _L_LO), _L_HI)
        h0b = jnp.minimum(jnp.maximum(h0b, _L_LO), _L_HI)

        cw1.wait()
        sb1 = jnp.where(t == 0, lb1, 0.0).astype(jnp.float32)
        q = [sb1 * w1s_v[0, pl.ds(16 * o, 16)] for o in range(8)]
        for j in range(32):
            s = h0a[j] if j < 16 else h0b[j - 16]
            for o in range(8):
                q[o] = q[o] + s * w1s_v[j + 1, pl.ds(16 * o, 16)]
        for o in range(8):
            part_v[pl.ds(16 * o, 16)] = q[o]
        pltpu.sync_copy(part_v, shr_h1.at[t])
        plsc.subcore_barrier()

        @pl.when(t == 0)
        def _final():
            pltpu.sync_copy(shr_h1, sh_v)
            for o in range(8):
                z = sh_v[0, pl.ds(16 * o, 16)]
                for u in range(1, _NS):
                    z = z + sh_v[u, pl.ds(16 * o, 16)]
                y = 1.0 / (1.0 + jnp.exp(-z))
                y = jnp.minimum(jnp.maximum(y, _EPS), 1.0 - _EPS)
                out_v[pl.ds(16 * o, 16)] = y
            pltpu.sync_copy(out_v, out_h)


@functools.cache
def _gln_kernel():
    mesh = plsc.VectorSubcoreMesh(core_axis_name="c", subcore_axis_name="s",
                                  num_cores=2, num_subcores=_NS)
    return pl.kernel(
        _gln_body,
        out_type=jax.ShapeDtypeStruct((_OUT,), jnp.float32),
        mesh=mesh,
        scratch_types=[
            pltpu.VMEM((_IN,), jnp.float32),
            pltpu.VMEM((_IN, 16), jnp.float32),
            pltpu.VMEM((16,), jnp.float32),
            pltpu.VMEM((16,), jnp.float32),
            pltpu.VMEM((136, 128), jnp.float32),
            pltpu.VMEM((40, _OUT), jnp.float32),
            pltpu.VMEM((4, _OUT), jnp.float32),
            pltpu.VMEM((_OUT,), jnp.float32),
            pltpu.VMEM((_OUT,), jnp.float32),
            pltpu.VMEM_SHARED((_NS, _OUT), jnp.float32),
            pltpu.VMEM_SHARED((_NS, _OUT), jnp.float32),
            pltpu.VMEM((_NS, _OUT), jnp.float32),
            pltpu.SemaphoreType.DMA,
            pltpu.SemaphoreType.DMA,
            pltpu.SemaphoreType.DMA,
        ],
    )


def kernel(side_info, offset0, normal0, bias0, W0, offset1, normal1, bias1, W1):
    nboth = jnp.concatenate([normal0, normal1], axis=1)
    offs = jnp.concatenate([offset0, offset1], axis=0)
    lb0 = jnp.log(bias0) - jnp.log1p(-bias0)
    lb1 = jnp.log(bias1) - jnp.log1p(-bias1)
    cons = jnp.concatenate(
        [lb0, lb1, jnp.zeros((14,), jnp.float32)], axis=0)
    return _gln_kernel()(nboth, side_info, offs, cons, W0, W1)


# speedup vs baseline: 1.0118x; 1.0118x over previous
import functools
import math

import jax
import jax.numpy as jnp
from jax import lax
from jax.experimental import pallas as pl
from jax.experimental.pallas import tpu as pltpu
from jax.experimental.pallas import tpu_sc as plsc

_EPS = 0.001
_L_LO = math.log(_EPS) - math.log1p(-_EPS)
_L_HI = -_L_LO

_IN = 512
_HID = 512
_OUT = 128
_NS = 16


def _gln_body(nboth_h, side_h, offs_h, cons_h, w0_h, w1_h, out_h,
              side_v, nb_v, offs_v, cons_v, w0s_v, w1s_v, tmp4_v,
              part_v, acc_v, out_v, shr_g, shr_h0, shr_h1,
              sem_in, sem_w0a, sem_w0b, sem_w1):
    cid = lax.axis_index("c")
    sid = lax.axis_index("s")

    @pl.when(cid == 0)
    def _core0():
        t = sid
        b = t >> 2
        r = t & 3
        g_r0 = pl.multiple_of(t * 32, 32)

        c1 = pltpu.async_copy(side_h, side_v, sem_in)
        c2 = pltpu.async_copy(nboth_h.at[pl.ds(g_r0, 32), :], nb_v, sem_in)
        c3 = pltpu.async_copy(offs_h, offs_v, sem_in)
        c4 = pltpu.async_copy(cons_h, cons_v, sem_in)
        c1.wait(); c2.wait(); c3.wait(); c4.wait()

        pacc = jnp.zeros((16,), jnp.float32)
        for k in range(2):
            sv = side_v[pl.ds(t * 32 + k * 16, 16)]
            for j in range(16):
                pacc = pacc + sv[j] * nb_v[k * 16 + j, :]
        part_v[pl.ds(0, 16)] = pacc
        pltpu.sync_copy(part_v, shr_g.at[t])
        plsc.subcore_barrier()
        pltpu.sync_copy(shr_g, acc_v)
        acc = acc_v[0, pl.ds(0, 16)]
        for u in range(1, _NS):
            acc = acc + acc_v[u, pl.ds(0, 16)]

        bits = acc > offs_v[...]
        lane = lax.iota(jnp.int32, 16)
        pw = jnp.int32(1) << (7 - (lane & 7))
        ivp = jnp.where(bits, pw, jnp.int32(0))
        g0 = ivp[0]
        g1 = ivp[8]
        for j in range(1, 8):
            g0 = g0 + ivp[j]
            g1 = g1 + ivp[8 + j]

        row0 = pl.multiple_of(r * 128, 128)
        col0 = pl.multiple_of(b * 128, 128)
        w1r0 = pl.multiple_of(t * 32, 32)
        cw0a = pltpu.async_copy(
            w0_h.at[g0, pl.ds(row0, 72), pl.ds(col0, 128)],
            w0s_v.at[pl.ds(0, 72), :], sem_w0a)
        cw0b = pltpu.async_copy(
            w0_h.at[g0, pl.ds(row0 + 72, 64), pl.ds(col0, 128)],
            w0s_v.at[pl.ds(72, 64), :], sem_w0b)
        cw1 = pltpu.async_copy(
            w1_h.at[g1, pl.ds(w1r0, 40), :], w1s_v, sem_w1)

        cv = cons_v[...]
        lb0 = cv[0]
        lb1 = cv[1]

        cw0a.wait()
        sb0 = jnp.where(r == 0, lb0, 0.0).astype(jnp.float32)
        p = [sb0 * w0s_v[0, pl.ds(16 * o, 16)] for o in range(8)]

        def mbody(k, carry):
            sv = side_v[pl.ds(r * 128 + k * 16, 16)]
            sv = jnp.minimum(jnp.maximum(sv, _EPS), 1.0 - _EPS)
            out = list(carry)
            for j in range(16):
                s = sv[j]
                l = k * 16 + j + 1
                for o in range(8):
                    out[o] = out[o] + s * w0s_v[l, pl.ds(16 * o, 16)]
            return tuple(out)
        p = lax.fori_loop(0, 4, mbody, tuple(p))
        cw0b.wait()
        p = lax.fori_loop(4, 8, mbody, p)

        for o in range(8):
            part_v[pl.ds(16 * o, 16)] = p[o]
        pltpu.sync_copy(part_v, shr_h0.at[t])
        plsc.subcore_barrier()

        pltpu.sync_copy(shr_h0.at[pl.ds(b * 4, 4), :], tmp4_v)
        c0 = r * 32
        h0a = (tmp4_v[0, pl.ds(c0, 16)] + tmp4_v[1, pl.ds(c0, 16)]
               + tmp4_v[2, pl.ds(c0, 16)] + tmp4_v[3, pl.ds(c0, 16)])
        c1_ = c0 + 16
        h0b = (tmp4_v[0, pl.ds(c1_, 16)] + tmp4_v[1, pl.ds(c1_, 16)]
               + tmp4_v[2, pl.ds(c1_, 16)] + tmp4_v[3, pl.ds(c1_, 16)])
        h0a = jnp.minimum(jnp.maximum(h0a, _L_LO), _L_HI)
        h0b = jnp.minimum(jnp.maximum(h0b, _L_LO), _L_HI)

        cw1.wait()
        sb1 = jnp.where(t == 0, lb1, 0.0).astype(jnp.float32)
        q = [sb1 * w1s_v[0, pl.ds(16 * o, 16)] for o in range(8)]
        for j in range(32):
            s = h0a[j] if j < 16 else h0b[j - 16]
            for o in range(8):
                q[o] = q[o] + s * w1s_v[j + 1, pl.ds(16 * o, 16)]
        for o in range(8):
            part_v[pl.ds(16 * o, 16)] = q[o]
        pltpu.sync_copy(part_v, shr_h1.at[t])
        plsc.subcore_barrier()

        @pl.when(t < 8)
        def _final():
            pltpu.sync_copy(shr_h1, acc_v)
            co = pl.multiple_of(t * 16, 16)
            z = acc_v[0, pl.ds(co, 16)]
            for u in range(1, _NS):
                z = z + acc_v[u, pl.ds(co, 16)]
            y = 1.0 / (1.0 + jnp.exp(-z))
            y = jnp.minimum(jnp.maximum(y, _EPS), 1.0 - _EPS)
            out_v[pl.ds(0, 16)] = y
            pltpu.sync_copy(out_v, out_h.at[pl.ds(co, 16)])


@functools.cache
def _gln_kernel():
    mesh = plsc.VectorSubcoreMesh(core_axis_name="c", subcore_axis_name="s",
                                  num_cores=2, num_subcores=_NS)
    return pl.kernel(
        _gln_body,
        out_type=jax.ShapeDtypeStruct((_OUT,), jnp.float32),
        mesh=mesh,
        scratch_types=[
            pltpu.VMEM((_IN,), jnp.float32),
            pltpu.VMEM((32, 16), jnp.float32),
            pltpu.VMEM((16,), jnp.float32),
            pltpu.VMEM((16,), jnp.float32),
            pltpu.VMEM((136, 128), jnp.float32),
            pltpu.VMEM((40, _OUT), jnp.float32),
            pltpu.VMEM((4, _OUT), jnp.float32),
            pltpu.VMEM((_OUT,), jnp.float32),
            pltpu.VMEM((_NS, _OUT), jnp.float32),
            pltpu.VMEM((16,), jnp.float32),
            pltpu.VMEM_SHARED((_NS, _OUT), jnp.float32),
            pltpu.VMEM_SHARED((_NS, _OUT), jnp.float32),
            pltpu.VMEM_SHARED((_NS, _OUT), jnp.float32),
            pltpu.SemaphoreType.DMA,
            pltpu.SemaphoreType.DMA,
            pltpu.SemaphoreType.DMA,
            pltpu.SemaphoreType.DMA,
        ],
    )


def kernel(side_info, offset0, normal0, bias0, W0, offset1, normal1, bias1, W1):
    nboth = jnp.concatenate([normal0, normal1], axis=1)
    offs = jnp.concatenate([offset0, offset1], axis=0)
    lb0 = jnp.log(bias0) - jnp.log1p(-bias0)
    lb1 = jnp.log(bias1) - jnp.log1p(-bias1)
    cons = jnp.concatenate(
        [lb0, lb1, jnp.zeros((14,), jnp.float32)], axis=0)
    return _gln_kernel()(nboth, side_info, offs, cons, W0, W1)


# speedup vs baseline: 10.2482x; 10.1292x over previous
import functools
import math

import jax
import jax.numpy as jnp
from jax import lax
from jax.experimental import pallas as pl
from jax.experimental.pallas import tpu as pltpu
from jax.experimental.pallas import tpu_sc as plsc

_EPS = 0.001
_L_LO = math.log(_EPS) - math.log1p(-_EPS)
_L_HI = -_L_LO

_IN = 512
_HID = 512
_OUT = 128
_NS = 16


_GDN = lax.GatherDimensionNumbers(
    offset_dims=(), collapsed_slice_dims=(0,), start_index_map=(0,))


def _bc(v, j):
    idx = jnp.full((16, 1), j, dtype=jnp.int32)
    return lax.gather(v, idx, _GDN, (1,),
                      mode=lax.GatherScatterMode.PROMISE_IN_BOUNDS)


def _gln_body(nboth_h, side_h, offs_h, cons_h, w0_h, w1_h, out_h,
              side_v, nb_v, offs_v, cons_v, w0s_v, w1s_v, tmp4_v,
              part_v, acc_v, out_v, shr_g, shr_h0, shr_h1,
              sem_in, sem_w0a, sem_w0b, sem_w1):
    cid = lax.axis_index("c")
    sid = lax.axis_index("s")

    @pl.when(cid == 0)
    def _core0():
        t = sid
        b = t >> 2
        r = t & 3
        g_r0 = pl.multiple_of(t * 32, 32)

        c1 = pltpu.async_copy(side_h, side_v, sem_in)
        g_rh = pl.multiple_of(t * 4, 4)
        c2 = pltpu.async_copy(nboth_h.at[pl.ds(g_rh, 4)], nb_v, sem_in)
        c3 = pltpu.async_copy(offs_h, offs_v, sem_in)
        c4 = pltpu.async_copy(cons_h, cons_v, sem_in)
        c1.wait(); c2.wait(); c3.wait(); c4.wait()

        pacc = jnp.zeros((16,), jnp.float32)
        for k in range(2):
            sv = side_v[pl.ds(t * 32 + k * 16, 16)]
            for j in range(16):
                pacc = pacc + _bc(sv, j) * nb_v[2 * k + (j >> 3), j & 7,
                                                pl.ds(0, 16)]
        part_v[pl.ds(0, 16)] = pacc
        pltpu.sync_copy(part_v, shr_g.at[t])
        plsc.subcore_barrier()
        pltpu.sync_copy(shr_g, acc_v)
        acc = acc_v[0, pl.ds(0, 16)]
        for u in range(1, _NS):
            acc = acc + acc_v[u, pl.ds(0, 16)]

        bits = acc > offs_v[...]
        lane = lax.iota(jnp.int32, 16)
        pw = jnp.int32(1) << (7 - (lane & 7))
        ivp = jnp.where(bits, pw, jnp.int32(0))
        g0 = ivp[0]
        g1 = ivp[8]
        for j in range(1, 8):
            g0 = g0 + ivp[j]
            g1 = g1 + ivp[8 + j]

        row0 = pl.multiple_of(r * 128, 128)
        w1r0 = pl.multiple_of(t * 32, 32)
        e0h = g0 >> 3
        e0l = g0 & 7
        e1h = g1 >> 3
        e1l = g1 & 7
        cw0a = pltpu.async_copy(
            w0_h.at[pl.ds(row0, 72), e0h, b, e0l, :],
            w0s_v.at[pl.ds(0, 72), :], sem_w0a)
        cw0b = pltpu.async_copy(
            w0_h.at[pl.ds(row0 + 72, 57), e0h, b, e0l, :],
            w0s_v.at[pl.ds(72, 57), :], sem_w0b)
        cw1 = pltpu.async_copy(
            w1_h.at[pl.ds(w1r0, 33), e1h, 0, e1l, :], w1s_v, sem_w1)

        cv = cons_v[...]
        lb0 = cv[0]
        lb1 = cv[1]

        cw0a.wait()
        sb0 = jnp.where(r == 0, lb0, 0.0).astype(jnp.float32)
        p = [sb0 * w0s_v[0, pl.ds(16 * o, 16)] for o in range(8)]

        def mbody(k, carry):
            sv = side_v[pl.ds(r * 128 + k * 16, 16)]
            sv = jnp.minimum(jnp.maximum(sv, _EPS), 1.0 - _EPS)
            out = list(carry)
            for j in range(16):
                s = _bc(sv, j)
                l = k * 16 + j + 1
                for o in range(8):
                    out[o] = out[o] + s * w0s_v[l, pl.ds(16 * o, 16)]
            return tuple(out)
        p = lax.fori_loop(0, 4, mbody, tuple(p))
        cw0b.wait()
        p = lax.fori_loop(4, 8, mbody, p)

        for o in range(8):
            part_v[pl.ds(16 * o, 16)] = p[o]
        pltpu.sync_copy(part_v, shr_h0.at[t])
        plsc.subcore_barrier()

        pltpu.sync_copy(shr_h0.at[pl.ds(b * 4, 4), :], tmp4_v)
        c0 = r * 32
        h0a = (tmp4_v[0, pl.ds(c0, 16)] + tmp4_v[1, pl.ds(c0, 16)]
               + tmp4_v[2, pl.ds(c0, 16)] + tmp4_v[3, pl.ds(c0, 16)])
        c1_ = c0 + 16
        h0b = (tmp4_v[0, pl.ds(c1_, 16)] + tmp4_v[1, pl.ds(c1_, 16)]
               + tmp4_v[2, pl.ds(c1_, 16)] + tmp4_v[3, pl.ds(c1_, 16)])
        h0a = jnp.minimum(jnp.maximum(h0a, _L_LO), _L_HI)
        h0b = jnp.minimum(jnp.maximum(h0b, _L_LO), _L_HI)

        cw1.wait()
        sb1 = jnp.where(t == 0, lb1, 0.0).astype(jnp.float32)
        q = [sb1 * w1s_v[0, pl.ds(16 * o, 16)] for o in range(8)]
        for j in range(32):
            s = _bc(h0a, j) if j < 16 else _bc(h0b, j - 16)
            for o in range(8):
                q[o] = q[o] + s * w1s_v[j + 1, pl.ds(16 * o, 16)]
        for o in range(8):
            part_v[pl.ds(16 * o, 16)] = q[o]
        pltpu.sync_copy(part_v, shr_h1.at[t])
        plsc.subcore_barrier()

        @pl.when(t < 8)
        def _final():
            pltpu.sync_copy(shr_h1, acc_v)
            co = pl.multiple_of(t * 16, 16)
            z = acc_v[0, pl.ds(co, 16)]
            for u in range(1, _NS):
                z = z + acc_v[u, pl.ds(co, 16)]
            y = 1.0 / (1.0 + jnp.exp(-z))
            y = jnp.minimum(jnp.maximum(y, _EPS), 1.0 - _EPS)
            out_v[pl.ds(0, 16)] = y
            pltpu.sync_copy(out_v, out_h.at[pl.ds(co, 16)])


@functools.cache
def _gln_kernel():
    mesh = plsc.VectorSubcoreMesh(core_axis_name="c", subcore_axis_name="s",
                                  num_cores=2, num_subcores=_NS)
    return pl.kernel(
        _gln_body,
        out_type=jax.ShapeDtypeStruct((_OUT,), jnp.float32),
        mesh=mesh,
        compiler_params=pltpu.CompilerParams(use_tc_tiling_on_sc=False),
        scratch_types=[
            pltpu.VMEM((_IN,), jnp.float32),
            pltpu.VMEM((4, 8, 128), jnp.float32),
            pltpu.VMEM((16,), jnp.float32),
            pltpu.VMEM((16,), jnp.float32),
            pltpu.VMEM((129, 128), jnp.float32),
            pltpu.VMEM((33, _OUT), jnp.float32),
            pltpu.VMEM((4, _OUT), jnp.float32),
            pltpu.VMEM((_OUT,), jnp.float32),
            pltpu.VMEM((_NS, _OUT), jnp.float32),
            pltpu.VMEM((16,), jnp.float32),
            pltpu.VMEM_SHARED((_NS, _OUT), jnp.float32),
            pltpu.VMEM_SHARED((_NS, _OUT), jnp.float32),
            pltpu.VMEM_SHARED((_NS, _OUT), jnp.float32),
            pltpu.SemaphoreType.DMA,
            pltpu.SemaphoreType.DMA,
            pltpu.SemaphoreType.DMA,
            pltpu.SemaphoreType.DMA,
        ],
    )


def kernel(side_info, offset0, normal0, bias0, W0, offset1, normal1, bias1, W1):
    nboth = jnp.concatenate([normal0, normal1], axis=1)
    nbp = jnp.zeros((64, 8, 128), jnp.float32)
    nbp = nbp.at[:, :, :16].set(nboth.reshape(64, 8, 16))
    offs = jnp.concatenate([offset0, offset1], axis=0)
    lb0 = jnp.log(bias0) - jnp.log1p(-bias0)
    lb1 = jnp.log(bias1) - jnp.log1p(-bias1)
    cons = jnp.concatenate(
        [lb0, lb1, jnp.zeros((14,), jnp.float32)], axis=0)
    w0w = W0.reshape(32, 8, 513, 4, 128).transpose(2, 0, 3, 1, 4)
    w1w = W1.reshape(32, 8, 513, 1, 128).transpose(2, 0, 3, 1, 4)
    return _gln_kernel()(nbp, side_info, offs, cons, w0w, w1w)


# speedup vs baseline: 11.3040x; 1.1030x over previous
import functools
import math

import jax
import jax.numpy as jnp
from jax import lax
from jax.experimental import pallas as pl
from jax.experimental.pallas import tpu as pltpu
from jax.experimental.pallas import tpu_sc as plsc

_EPS = 0.001
_L_LO = math.log(_EPS) - math.log1p(-_EPS)
_L_HI = -_L_LO

_IN = 512
_OUT = 128
_NS = 16

_GDN = lax.GatherDimensionNumbers(
    offset_dims=(), collapsed_slice_dims=(0,), start_index_map=(0,))


def _bc(v, j):
    idx = jnp.full((16, 1), j, dtype=jnp.int32)
    return lax.gather(v, idx, _GDN, (1,),
                      mode=lax.GatherScatterMode.PROMISE_IN_BOUNDS)


def _rot(v, s):
    idx = ((lax.iota(jnp.int32, 16) + s) & 15).reshape(16, 1)
    return lax.gather(v, idx, _GDN, (1,),
                      mode=lax.GatherScatterMode.PROMISE_IN_BOUNDS)


def _lsum(v):
    for s in (8, 4, 2, 1):
        v = v + _rot(v, s)
    return v


def _gln_body(n0_h, n1_h, side_h, off0_h, off1_h, cons_h, w0_h, w1_h, out_h,
              side_v, n0_v, n1_v, off_v, cons_v, w0s_v, w1s_v, tmp4_v,
              part_v, acc_v, out_v, shr_h0, shr_h1,
              sem_in, sem_w0a, sem_w0b, sem_w1):
    cid = lax.axis_index("c")
    sid = lax.axis_index("s")

    @pl.when(cid == 0)
    def _core0():
        t = sid
        b = t >> 2
        r = t & 3

        c1 = pltpu.async_copy(side_h, side_v, sem_in)
        c2 = pltpu.async_copy(n0_h, n0_v, sem_in)
        c3 = pltpu.async_copy(n1_h, n1_v, sem_in)
        c4 = pltpu.async_copy(off0_h, off_v.at[pl.ds(0, 8)], sem_in)
        c5 = pltpu.async_copy(off1_h, off_v.at[pl.ds(8, 8)], sem_in)
        c6 = pltpu.async_copy(cons_h, cons_v, sem_in)
        c1.wait(); c2.wait(); c3.wait(); c4.wait(); c5.wait(); c6.wait()

        def gbody(k, carry):
            ch = k >> 3
            lo = (k & 7) * 16
            sv = side_v[pl.ds(k * 16, 16)]
            out = list(carry)
            for j in range(8):
                out[j] = out[j] + sv * n0_v[ch, j, pl.ds(lo, 16)]
                out[8 + j] = out[8 + j] + sv * n1_v[ch, j, pl.ds(lo, 16)]
            return tuple(out)
        zero = jnp.zeros((16,), jnp.float32)
        accs = lax.fori_loop(0, 32, gbody, (zero,) * 16)

        offv = off_v[...]
        g0 = jnp.int32(0)
        g1 = jnp.int32(0)
        for j in range(8):
            g0 = g0 + jnp.where(_lsum(accs[j])[0] > offv[j],
                                jnp.int32(1 << (7 - j)), jnp.int32(0))
            g1 = g1 + jnp.where(_lsum(accs[8 + j])[0] > offv[8 + j],
                                jnp.int32(1 << (7 - j)), jnp.int32(0))

        row0 = pl.multiple_of(r * 128, 128)
        w1r0 = pl.multiple_of(t * 32, 32)
        cw0a = pltpu.async_copy(
            w0_h.at[pl.ds(row0, 72), g0 >> 3, b, g0 & 7, :],
            w0s_v.at[pl.ds(0, 72), :], sem_w0a)
        cw0b = pltpu.async_copy(
            w0_h.at[pl.ds(row0 + 72, 57), g0 >> 3, b, g0 & 7, :],
            w0s_v.at[pl.ds(72, 57), :], sem_w0b)
        cw1 = pltpu.async_copy(
            w1_h.at[pl.ds(w1r0, 33), g1 >> 3, 0, g1 & 7, :], w1s_v, sem_w1)

        cv = cons_v[...]
        lb0 = cv[0]
        lb1 = cv[1]

        cw0a.wait()
        sb0 = jnp.where(r == 0, lb0, 0.0).astype(jnp.float32)
        p = [sb0 * w0s_v[0, pl.ds(16 * o, 16)] for o in range(8)]

        def mouter(k, carry):
            sv = side_v[pl.ds(r * 128 + k * 16, 16)]
            sv = jnp.minimum(jnp.maximum(sv, _EPS), 1.0 - _EPS)

            def minner(j, inner):
                s = _bc(sv, j)
                l = k * 16 + j + 1
                return tuple(inner[o] + s * w0s_v[l, pl.ds(16 * o, 16)]
                             for o in range(8))
            return lax.fori_loop(0, 16, minner, carry)
        p = lax.fori_loop(0, 4, mouter, tuple(p))
        cw0b.wait()
        p = lax.fori_loop(4, 8, mouter, p)

        for o in range(8):
            part_v[pl.ds(16 * o, 16)] = p[o]
        pltpu.sync_copy(part_v, shr_h0.at[t])
        plsc.subcore_barrier()

        pltpu.sync_copy(shr_h0.at[pl.ds(b * 4, 4), :], tmp4_v)
        c0 = r * 32
        h0a = (tmp4_v[0, pl.ds(c0, 16)] + tmp4_v[1, pl.ds(c0, 16)]
               + tmp4_v[2, pl.ds(c0, 16)] + tmp4_v[3, pl.ds(c0, 16)])
        c1_ = c0 + 16
        h0b = (tmp4_v[0, pl.ds(c1_, 16)] + tmp4_v[1, pl.ds(c1_, 16)]
               + tmp4_v[2, pl.ds(c1_, 16)] + tmp4_v[3, pl.ds(c1_, 16)])
        h0a = jnp.minimum(jnp.maximum(h0a, _L_LO), _L_HI)
        h0b = jnp.minimum(jnp.maximum(h0b, _L_LO), _L_HI)

        cw1.wait()
        sb1 = jnp.where(t == 0, lb1, 0.0).astype(jnp.float32)
        q = [sb1 * w1s_v[0, pl.ds(16 * o, 16)] for o in range(8)]

        def l1body(j, carry):
            s = jnp.where(j < 16, _bc(h0a, j & 15), _bc(h0b, j & 15))
            return tuple(carry[o] + s * w1s_v[j + 1, pl.ds(16 * o, 16)]
                         for o in range(8))
        q = lax.fori_loop(0, 32, l1body, tuple(q))
        for o in range(8):
            part_v[pl.ds(16 * o, 16)] = q[o]
        pltpu.sync_copy(part_v, shr_h1.at[t])
        plsc.subcore_barrier()

        @pl.when(t < 8)
        def _final():
            pltpu.sync_copy(shr_h1, acc_v)
            co = pl.multiple_of(t * 16, 16)

            def rbody(u, z):
                return z + acc_v[u, pl.ds(co, 16)]
            z = lax.fori_loop(1, _NS, rbody, acc_v[0, pl.ds(co, 16)])
            y = 1.0 / (1.0 + jnp.exp(-z))
            y = jnp.minimum(jnp.maximum(y, _EPS), 1.0 - _EPS)
            out_v[...] = y
            pltpu.sync_copy(out_v, out_h.at[pl.ds(co, 16)])


@functools.cache
def _gln_kernel():
    mesh = plsc.VectorSubcoreMesh(core_axis_name="c", subcore_axis_name="s",
                                  num_cores=2, num_subcores=_NS)
    return pl.kernel(
        _gln_body,
        out_type=jax.ShapeDtypeStruct((_OUT,), jnp.float32),
        mesh=mesh,
        compiler_params=pltpu.CompilerParams(use_tc_tiling_on_sc=False),
        scratch_types=[
            pltpu.VMEM((_IN,), jnp.float32),
            pltpu.VMEM((4, 8, 128), jnp.float32),
            pltpu.VMEM((4, 8, 128), jnp.float32),
            pltpu.VMEM((16,), jnp.float32),
            pltpu.VMEM((16,), jnp.float32),
            pltpu.VMEM((129, 128), jnp.float32),
            pltpu.VMEM((33, _OUT), jnp.float32),
            pltpu.VMEM((4, _OUT), jnp.float32),
            pltpu.VMEM((_OUT,), jnp.float32),
            pltpu.VMEM((_NS, _OUT), jnp.float32),
            pltpu.VMEM((16,), jnp.float32),
            pltpu.VMEM_SHARED((_NS, _OUT), jnp.float32),
            pltpu.VMEM_SHARED((_NS, _OUT), jnp.float32),
            pltpu.SemaphoreType.DMA,
            pltpu.SemaphoreType.DMA,
            pltpu.SemaphoreType.DMA,
            pltpu.SemaphoreType.DMA,
        ],
    )


def kernel(side_info, offset0, normal0, bias0, W0, offset1, normal1, bias1, W1):
    lb0 = jnp.log(bias0) - jnp.log1p(-bias0)
    lb1 = jnp.log(bias1) - jnp.log1p(-bias1)
    cons = jnp.concatenate(
        [lb0, lb1, jnp.zeros((14,), jnp.float32)], axis=0)
    w0w = W0.reshape(32, 8, 513, 4, 128).transpose(2, 0, 3, 1, 4)
    w1w = W1.reshape(32, 8, 513, 1, 128).transpose(2, 0, 3, 1, 4)
    n0w = normal0.T.reshape(8, 4, 128).transpose(1, 0, 2)
    n1w = normal1.T.reshape(8, 4, 128).transpose(1, 0, 2)
    return _gln_kernel()(n0w, n1w, side_info, offset0, offset1, cons,
                         w0w, w1w)


# speedup vs baseline: 11.9609x; 1.0581x over previous
import functools
import math

import jax
import jax.numpy as jnp
from jax import lax
from jax.experimental import pallas as pl
from jax.experimental.pallas import tpu as pltpu
from jax.experimental.pallas import tpu_sc as plsc

_EPS = 0.001
_L_LO = math.log(_EPS) - math.log1p(-_EPS)
_L_HI = -_L_LO

_IN = 512
_OUT = 128
_NS = 16

_GDN = lax.GatherDimensionNumbers(
    offset_dims=(), collapsed_slice_dims=(0,), start_index_map=(0,))


def _bc(v, j):
    idx = jnp.full((16, 1), j, dtype=jnp.int32)
    return lax.gather(v, idx, _GDN, (1,),
                      mode=lax.GatherScatterMode.PROMISE_IN_BOUNDS)


def _rot(v, s):
    idx = ((lax.iota(jnp.int32, 16) + s) & 15).reshape(16, 1)
    return lax.gather(v, idx, _GDN, (1,),
                      mode=lax.GatherScatterMode.PROMISE_IN_BOUNDS)


def _lsum(v):
    for s in (8, 4, 2, 1):
        v = v + _rot(v, s)
    return v


def _gln_body(n0_h, n1_h, side_h, off0_h, off1_h, cons_h, w0_h, w1_h, out_h,
              side_v, n0_v, n1_v, off_v, cons_v, w0s_v, w1s_v, tmp4_v,
              part_v, acc_v, out_v, shr_h0, shr_h1,
              sem_in, sem_w0a, sem_w0b, sem_w1):
    sid = lax.axis_index("s")

    @pl.when(sid >= 0)
    def _core0():
        t = sid
        b = t >> 2
        r = t & 3

        c1 = pltpu.async_copy(side_h, side_v, sem_in)
        c2 = pltpu.async_copy(n0_h, n0_v, sem_in)
        c3 = pltpu.async_copy(n1_h, n1_v, sem_in)
        c4 = pltpu.async_copy(off0_h, off_v.at[pl.ds(0, 8)], sem_in)
        c5 = pltpu.async_copy(off1_h, off_v.at[pl.ds(8, 8)], sem_in)
        c6 = pltpu.async_copy(cons_h, cons_v, sem_in)
        c1.wait(); c2.wait(); c3.wait(); c4.wait(); c5.wait(); c6.wait()

        def gbody(k, carry):
            ch = k >> 3
            lo = (k & 7) * 16
            sv = side_v[pl.ds(k * 16, 16)]
            out = list(carry)
            for j in range(8):
                out[j] = out[j] + sv * n0_v[ch, j, pl.ds(lo, 16)]
                out[8 + j] = out[8 + j] + sv * n1_v[ch, j, pl.ds(lo, 16)]
            return tuple(out)
        zero = jnp.zeros((16,), jnp.float32)
        accs = lax.fori_loop(0, 32, gbody, (zero,) * 16)

        offv = off_v[...]
        g0 = jnp.int32(0)
        g1 = jnp.int32(0)
        for j in range(8):
            g0 = g0 + jnp.where(_lsum(accs[j])[0] > offv[j],
                                jnp.int32(1 << (7 - j)), jnp.int32(0))
            g1 = g1 + jnp.where(_lsum(accs[8 + j])[0] > offv[8 + j],
                                jnp.int32(1 << (7 - j)), jnp.int32(0))

        row0 = pl.multiple_of(r * 128, 128)
        w1r0 = pl.multiple_of(t * 32, 32)
        cw0a = pltpu.async_copy(
            w0_h.at[pl.ds(row0, 72), g0 >> 3, b, g0 & 7, :],
            w0s_v.at[pl.ds(0, 72), :], sem_w0a)
        cw0b = pltpu.async_copy(
            w0_h.at[pl.ds(row0 + 72, 57), g0 >> 3, b, g0 & 7, :],
            w0s_v.at[pl.ds(72, 57), :], sem_w0b)
        cw1 = pltpu.async_copy(
            w1_h.at[pl.ds(w1r0, 33), g1 >> 3, 0, g1 & 7, :], w1s_v, sem_w1)

        cv = cons_v[...]
        lb0 = cv[0]
        lb1 = cv[1]

        cw0a.wait()
        sb0 = jnp.where(r == 0, lb0, 0.0).astype(jnp.float32)
        p = [sb0 * w0s_v[0, pl.ds(16 * o, 16)] for o in range(8)]

        def mouter(k, carry):
            sv = side_v[pl.ds(r * 128 + k * 16, 16)]
            sv = jnp.minimum(jnp.maximum(sv, _EPS), 1.0 - _EPS)

            def minner(j, inner):
                s = _bc(sv, j)
                l = k * 16 + j + 1
                return tuple(inner[o] + s * w0s_v[l, pl.ds(16 * o, 16)]
                             for o in range(8))
            return lax.fori_loop(0, 16, minner, carry)
        p = lax.fori_loop(0, 4, mouter, tuple(p))
        cw0b.wait()
        p = lax.fori_loop(4, 8, mouter, p)

        for o in range(8):
            part_v[pl.ds(16 * o, 16)] = p[o]
        pltpu.sync_copy(part_v, shr_h0.at[t])
        plsc.subcore_barrier()

        pltpu.sync_copy(shr_h0.at[pl.ds(b * 4, 4), :], tmp4_v)
        c0 = r * 32
        h0a = (tmp4_v[0, pl.ds(c0, 16)] + tmp4_v[1, pl.ds(c0, 16)]
               + tmp4_v[2, pl.ds(c0, 16)] + tmp4_v[3, pl.ds(c0, 16)])
        c1_ = c0 + 16
        h0b = (tmp4_v[0, pl.ds(c1_, 16)] + tmp4_v[1, pl.ds(c1_, 16)]
               + tmp4_v[2, pl.ds(c1_, 16)] + tmp4_v[3, pl.ds(c1_, 16)])
        h0a = jnp.minimum(jnp.maximum(h0a, _L_LO), _L_HI)
        h0b = jnp.minimum(jnp.maximum(h0b, _L_LO), _L_HI)

        cw1.wait()
        sb1 = jnp.where(t == 0, lb1, 0.0).astype(jnp.float32)
        q = [sb1 * w1s_v[0, pl.ds(16 * o, 16)] for o in range(8)]

        def l1body(j, carry):
            s = jnp.where(j < 16, _bc(h0a, j & 15), _bc(h0b, j & 15))
            return tuple(carry[o] + s * w1s_v[j + 1, pl.ds(16 * o, 16)]
                         for o in range(8))
        q = lax.fori_loop(0, 32, l1body, tuple(q))
        for o in range(8):
            part_v[pl.ds(16 * o, 16)] = q[o]
        pltpu.sync_copy(part_v, shr_h1.at[t])
        plsc.subcore_barrier()

        @pl.when(t < 8)
        def _final():
            pltpu.sync_copy(shr_h1, acc_v)
            co = pl.multiple_of(t * 16, 16)

            def rbody(u, z):
                return z + acc_v[u, pl.ds(co, 16)]
            z = lax.fori_loop(1, _NS, rbody, acc_v[0, pl.ds(co, 16)])
            y = 1.0 / (1.0 + jnp.exp(-z))
            y = jnp.minimum(jnp.maximum(y, _EPS), 1.0 - _EPS)
            out_v[...] = y
            pltpu.sync_copy(out_v, out_h.at[pl.ds(co, 16)])


@functools.cache
def _gln_kernel():
    mesh = plsc.VectorSubcoreMesh(core_axis_name="c", subcore_axis_name="s",
                                  num_cores=1, num_subcores=_NS)
    return pl.kernel(
        _gln_body,
        out_type=jax.ShapeDtypeStruct((_OUT,), jnp.float32),
        mesh=mesh,
        compiler_params=pltpu.CompilerParams(use_tc_tiling_on_sc=False, skip_device_barrier=True),
        scratch_types=[
            pltpu.VMEM((_IN,), jnp.float32),
            pltpu.VMEM((4, 8, 128), jnp.float32),
            pltpu.VMEM((4, 8, 128), jnp.float32),
            pltpu.VMEM((16,), jnp.float32),
            pltpu.VMEM((16,), jnp.float32),
            pltpu.VMEM((129, 128), jnp.float32),
            pltpu.VMEM((33, _OUT), jnp.float32),
            pltpu.VMEM((4, _OUT), jnp.float32),
            pltpu.VMEM((_OUT,), jnp.float32),
            pltpu.VMEM((_NS, _OUT), jnp.float32),
            pltpu.VMEM((16,), jnp.float32),
            pltpu.VMEM_SHARED((_NS, _OUT), jnp.float32),
            pltpu.VMEM_SHARED((_NS, _OUT), jnp.float32),
            pltpu.SemaphoreType.DMA,
            pltpu.SemaphoreType.DMA,
            pltpu.SemaphoreType.DMA,
            pltpu.SemaphoreType.DMA,
        ],
    )


def kernel(side_info, offset0, normal0, bias0, W0, offset1, normal1, bias1, W1):
    lb0 = jnp.log(bias0) - jnp.log1p(-bias0)
    lb1 = jnp.log(bias1) - jnp.log1p(-bias1)
    cons = jnp.concatenate(
        [lb0, lb1, jnp.zeros((14,), jnp.float32)], axis=0)
    w0w = W0.reshape(32, 8, 513, 4, 128).transpose(2, 0, 3, 1, 4)
    w1w = W1.reshape(32, 8, 513, 1, 128).transpose(2, 0, 3, 1, 4)
    n0w = normal0.T.reshape(8, 4, 128).transpose(1, 0, 2)
    n1w = normal1.T.reshape(8, 4, 128).transpose(1, 0, 2)
    return _gln_kernel()(n0w, n1w, side_info, offset0, offset1, cons,
                         w0w, w1w)
